# baseline (speedup 1.0000x reference)
import functools

import jax
import jax.numpy as jnp
import numpy as np
from jax import lax
from jax.experimental import pallas as pl
from jax.experimental.pallas import tpu as pltpu
from jax.experimental.pallas import tpu_sc as plsc

N_NODES = 10000
N_EDGES = 320000
C_IN = 16
C_OUT = 16
NBASIS = 16
HIDDEN = 256
ACT_NORM = 1.7130613088607788

NCHUNK = 2
EC = N_EDGES // NCHUNK

NC = 2
NS = 16
NW = NC * NS
EPW = EC // NW
CH = 40
NCH = EPW // CH
ROWS_PER_TILE = N_NODES // NS

B_TC = 3200
GRID_TC = EC // B_TC
PACK = 128 // C_IN
E8C = EC // PACK
B8 = B_TC // PACK


def _tc_body(es_ref, x1_ref, attr_ref, w1_ref, w2_ref, r_ref, s_ref, a_ref,
             out_ref):
    w1 = w1_ref[...]
    w2 = w2_ref[...]
    r = r_ref[...]
    s = s_ref[...]
    attr_b = jnp.dot(attr_ref[...], a_ref[...],
                     preferred_element_type=jnp.float32)
    for p in range(PACK):
        sl = pl.ds(p * C_IN, C_IN)
        h = jnp.dot(es_ref[:, sl], w1, preferred_element_type=jnp.float32)
        h = jnp.maximum(h, 0.0) * ACT_NORM
        wt = jnp.dot(h.astype(jnp.bfloat16), w2,
                     preferred_element_type=jnp.float32)
        x1b = jnp.dot(x1_ref[:, sl], r, preferred_element_type=jnp.float32)
        ef = jnp.dot(wt * x1b, s, preferred_element_type=jnp.float32)
        out_ref[:, sl] = ef * attr_b[:, p * C_IN:(p + 1) * C_IN]


def _tc_call(es8, x18, attr8p, w1s, w2s, r, s, a):
    return pl.pallas_call(
        _tc_body,
        grid=(GRID_TC,),
        in_specs=[
            pl.BlockSpec((B8, 128), lambda i: (i, 0)),
            pl.BlockSpec((B8, 128), lambda i: (i, 0)),
            pl.BlockSpec((B8, PACK), lambda i: (i, 0)),
            pl.BlockSpec((NBASIS, HIDDEN), lambda i: (0, 0)),
            pl.BlockSpec((HIDDEN, HIDDEN), lambda i: (0, 0)),
            pl.BlockSpec((C_IN, HIDDEN), lambda i: (0, 0)),
            pl.BlockSpec((HIDDEN, C_OUT), lambda i: (0, 0)),
            pl.BlockSpec((PACK, 128), lambda i: (0, 0)),
        ],
        out_specs=pl.BlockSpec((B8, 128), lambda i: (i, 0)),
        out_shape=jax.ShapeDtypeStruct((E8C, 128), jnp.float32),
    )(es8, x18, attr8p, w1s, w2s, r, s, a)


def _gather_x1(node_features, src3):
    mesh = plsc.VectorSubcoreMesh(core_axis_name="c", subcore_axis_name="s")

    @functools.partial(
        pl.kernel, mesh=mesh,
        out_type=jax.ShapeDtypeStruct((EC, C_IN), jnp.float32),
        scratch_types=[
            pltpu.VMEM((NCH, CH), jnp.int32),
            pltpu.VMEM((CH, C_IN), jnp.float32),
            pltpu.SemaphoreType.DMA,
        ],
        compiler_params=pltpu.CompilerParams(use_tc_tiling_on_sc=False),
    )
    def gk(table_hbm, src_hbm, out_hbm, idx_v, rows_v, sem):
        wid = lax.axis_index("s") * NC + lax.axis_index("c")
        base = wid * EPW
        pltpu.sync_copy(src_hbm.at[wid], idx_v)

        def body(j, carry):
            pltpu.async_copy(table_hbm.at[idx_v.at[j]], rows_v, sem).wait()
            pltpu.sync_copy(rows_v, out_hbm.at[pl.ds(base + j * CH, CH)])
            return carry

        lax.fori_loop(0, NCH, body, 0)

    return gk(node_features, src3)


def _scatter_out(ef2, dst3, init):
    mesh = plsc.VectorSubcoreMesh(core_axis_name="c", subcore_axis_name="s")

    @functools.partial(
        pl.kernel, mesh=mesh,
        out_type=jax.ShapeDtypeStruct((NC, N_NODES, C_OUT), jnp.float32),
        scratch_types=[
            pltpu.VMEM((NCH, CH), jnp.int32),
            pltpu.VMEM((CH, C_OUT), jnp.float32),
            pltpu.VMEM_SHARED((N_NODES, C_OUT), jnp.float32),
            pltpu.SemaphoreType.DMA,
        ],
        compiler_params=pltpu.CompilerParams(use_tc_tiling_on_sc=False),
    )
    def sk(ef_hbm, dst_hbm, z_hbm, out_hbm, idx_v, rows_v, acc, sem):
        c = lax.axis_index("c")
        s = lax.axis_index("s")
        wid = s * NC + c
        base = wid * EPW
        r0 = s * ROWS_PER_TILE
        pltpu.sync_copy(z_hbm.at[c, pl.ds(r0, ROWS_PER_TILE)],
                        acc.at[pl.ds(r0, ROWS_PER_TILE)])
        pltpu.sync_copy(dst_hbm.at[wid], idx_v)
        plsc.subcore_barrier()

        def body(j, carry):
            pltpu.async_copy(ef_hbm.at[pl.ds(base + j * CH, CH)],
                             rows_v, sem).wait()
            pltpu.sync_copy(rows_v, acc.at[idx_v.at[j]], add=True)
            return carry

        lax.fori_loop(0, NCH, body, 0)
        plsc.subcore_barrier()
        pltpu.sync_copy(acc.at[pl.ds(r0, ROWS_PER_TILE)],
                        out_hbm.at[c, pl.ds(r0, ROWS_PER_TILE)])

    return sk(ef2, dst3, init)


def _combine_body(p_ref, o_ref):
    o_ref[...] = p_ref[0] + p_ref[1]


def _combine(partials):
    return pl.pallas_call(
        _combine_body,
        out_shape=jax.ShapeDtypeStruct((N_NODES, C_OUT), jnp.float32),
    )(partials)


def kernel(node_features, edge_src, edge_dst, edge_attr, edge_scalars,
           num_neighbors, W1, W2):
    nn = jnp.asarray(num_neighbors, jnp.float32)
    w1s = W1 * np.float32(1.0 / np.sqrt(NBASIS))
    w2s = (W2 * (np.float32(1.0 / (np.sqrt(HIDDEN) * np.sqrt(C_IN)))
                 / jnp.sqrt(nn))).astype(jnp.bfloat16)
    r = jnp.repeat(jnp.eye(C_IN, dtype=jnp.float32), C_OUT, axis=1)
    s = jnp.tile(jnp.eye(C_OUT, dtype=jnp.float32), (C_IN, 1))
    a = jnp.repeat(jnp.eye(PACK, dtype=jnp.float32), C_IN, axis=1)

    src4 = edge_src.reshape(NCHUNK, NW, NCH, CH)
    dst4 = edge_dst.reshape(NCHUNK, NW, NCH, CH)
    es8 = edge_scalars.reshape(NCHUNK, E8C, 128)
    attr8p = edge_attr.reshape(NCHUNK, E8C, PACK)

    efc = []
    for cix in range(NCHUNK):
        x18 = _gather_x1(node_features, src4[cix]).reshape(E8C, 128)
        efc.append(_tc_call(es8[cix], x18, attr8p[cix], w1s, w2s, r, s, a)
                   .reshape(EC, C_OUT))
    acc = jnp.zeros((NC, N_NODES, C_OUT), jnp.float32)
    for cix in range(NCHUNK):
        acc = _scatter_out(efc[cix], dst4[cix], acc)
    return _combine(acc)


# speedup vs baseline: 1.2679x; 1.2679x over previous
import functools

import jax
import jax.numpy as jnp
import numpy as np
from jax import lax
from jax.experimental import pallas as pl
from jax.experimental.pallas import tpu as pltpu
from jax.experimental.pallas import tpu_sc as plsc

N_NODES = 10000
N_EDGES = 320000
C_IN = 16
C_OUT = 16
NBASIS = 16
HIDDEN = 256
ACT_NORM = 1.7130613088607788

NC = 2
NS = 16
NW = NC * NS
EPW = N_EDGES // NW
CH = 80
NCH = EPW // CH
GROUP = 5
NG = NCH // GROUP
ROWS_PER_TILE = N_NODES // NS

B_TC = 2560
GRID_TC = N_EDGES // B_TC
PACK = 128 // C_IN
E8 = N_EDGES // PACK
B8 = B_TC // PACK


def _tc_body(es_ref, x1_ref, attr_ref, w1_ref, w2_ref, r_ref, s_ref, a_ref,
             out_ref):
    w1 = w1_ref[...]
    w2 = w2_ref[...]
    r = r_ref[...]
    s = s_ref[...]
    attr_b = jnp.dot(attr_ref[...], a_ref[...],
                     preferred_element_type=jnp.float32)
    for p in range(PACK):
        sl = pl.ds(p * C_IN, C_IN)
        h = jnp.dot(es_ref[:, sl], w1, preferred_element_type=jnp.float32)
        h = jnp.maximum(h, 0.0) * ACT_NORM
        wt = jnp.dot(h.astype(jnp.bfloat16), w2,
                     preferred_element_type=jnp.float32)
        x1b = jnp.dot(x1_ref[:, sl], r, preferred_element_type=jnp.float32)
        ef = jnp.dot(wt * x1b, s, preferred_element_type=jnp.float32)
        out_ref[:, sl] = ef * attr_b[:, p * C_IN:(p + 1) * C_IN]


def _tc_call(es8, x18, attr8p, w1s, w2s, r, s, a):
    return pl.pallas_call(
        _tc_body,
        grid=(GRID_TC,),
        in_specs=[
            pl.BlockSpec((B8, 128), lambda i: (i, 0)),
            pl.BlockSpec((B8, 128), lambda i: (i, 0)),
            pl.BlockSpec((B8, PACK), lambda i: (i, 0)),
            pl.BlockSpec((NBASIS, HIDDEN), lambda i: (0, 0)),
            pl.BlockSpec((HIDDEN, HIDDEN), lambda i: (0, 0)),
            pl.BlockSpec((C_IN, HIDDEN), lambda i: (0, 0)),
            pl.BlockSpec((HIDDEN, C_OUT), lambda i: (0, 0)),
            pl.BlockSpec((PACK, 128), lambda i: (0, 0)),
        ],
        out_specs=pl.BlockSpec((B8, 128), lambda i: (i, 0)),
        out_shape=jax.ShapeDtypeStruct((E8, 128), jnp.float32),
    )(es8, x18, attr8p, w1s, w2s, r, s, a)


def _gather_x1(node_features, src3):
    mesh = plsc.VectorSubcoreMesh(core_axis_name="c", subcore_axis_name="s")

    @functools.partial(
        pl.kernel, mesh=mesh,
        out_type=jax.ShapeDtypeStruct((N_EDGES, C_IN), jnp.float32),
        scratch_types=[
            pltpu.VMEM((NCH, CH), jnp.int32),
            pltpu.VMEM((GROUP, CH, C_IN), jnp.float32),
            pltpu.SemaphoreType.DMA,
            pltpu.SemaphoreType.DMA,
        ],
        compiler_params=pltpu.CompilerParams(use_tc_tiling_on_sc=False),
    )
    def gk(table_hbm, src_hbm, out_hbm, idx_v, rows_v, gsem, wsem):
        wid = lax.axis_index("s") * NC + lax.axis_index("c")
        base = wid * EPW
        pltpu.sync_copy(src_hbm.at[wid], idx_v)

        def body(g, carry):
            j0 = g * GROUP
            cps = [
                pltpu.async_copy(table_hbm.at[idx_v.at[j0 + b]],
                                 rows_v.at[b], gsem)
                for b in range(GROUP)
            ]
            for cp in cps:
                cp.wait()
            wps = [
                pltpu.async_copy(
                    rows_v.at[b],
                    out_hbm.at[pl.ds(base + (j0 + b) * CH, CH)], wsem)
                for b in range(GROUP)
            ]
            for cp in wps:
                cp.wait()
            return carry

        lax.fori_loop(0, NG, body, 0)

    return gk(node_features, src3)


def _scatter_out(ef2, dst3, zeros):
    mesh = plsc.VectorSubcoreMesh(core_axis_name="c", subcore_axis_name="s")

    @functools.partial(
        pl.kernel, mesh=mesh,
        out_type=jax.ShapeDtypeStruct((NC, N_NODES, C_OUT), jnp.float32),
        scratch_types=[
            pltpu.VMEM((NCH, CH), jnp.int32),
            pltpu.VMEM((GROUP, CH, C_OUT), jnp.float32),
            pltpu.VMEM_SHARED((N_NODES, C_OUT), jnp.float32),
            pltpu.SemaphoreType.DMA,
        ],
        compiler_params=pltpu.CompilerParams(use_tc_tiling_on_sc=False),
    )
    def sk(ef_hbm, dst_hbm, z_hbm, out_hbm, idx_v, rows_v, acc, sem):
        c = lax.axis_index("c")
        s = lax.axis_index("s")
        wid = s * NC + c
        base = wid * EPW
        r0 = s * ROWS_PER_TILE
        pltpu.sync_copy(z_hbm.at[pl.ds(r0, ROWS_PER_TILE)],
                        acc.at[pl.ds(r0, ROWS_PER_TILE)])
        pltpu.sync_copy(dst_hbm.at[wid], idx_v)
        plsc.subcore_barrier()

        def body(g, carry):
            j0 = g * GROUP
            cps = [
                pltpu.async_copy(
                    ef_hbm.at[pl.ds(base + (j0 + b) * CH, CH)],
                    rows_v.at[b], sem)
                for b in range(GROUP)
            ]
            for cp in cps:
                cp.wait()
            for b in range(GROUP):
                pltpu.sync_copy(rows_v.at[b], acc.at[idx_v.at[j0 + b]],
                                add=True)
            return carry

        lax.fori_loop(0, NG, body, 0)
        plsc.subcore_barrier()
        pltpu.sync_copy(acc.at[pl.ds(r0, ROWS_PER_TILE)],
                        out_hbm.at[c, pl.ds(r0, ROWS_PER_TILE)])

    return sk(ef2, dst3, zeros)


def _combine_body(p_ref, o_ref):
    o_ref[...] = p_ref[0] + p_ref[1]


def _combine(partials):
    return pl.pallas_call(
        _combine_body,
        out_shape=jax.ShapeDtypeStruct((N_NODES, C_OUT), jnp.float32),
    )(partials)


def kernel(node_features, edge_src, edge_dst, edge_attr, edge_scalars,
           num_neighbors, W1, W2):
    nn = jnp.asarray(num_neighbors, jnp.float32)
    w1s = W1 * np.float32(1.0 / np.sqrt(NBASIS))
    w2s = (W2 * (np.float32(1.0 / (np.sqrt(HIDDEN) * np.sqrt(C_IN)))
                 / jnp.sqrt(nn))).astype(jnp.bfloat16)
    r = jnp.repeat(jnp.eye(C_IN, dtype=jnp.float32), C_OUT, axis=1)
    s = jnp.tile(jnp.eye(C_OUT, dtype=jnp.float32), (C_IN, 1))
    a = jnp.repeat(jnp.eye(PACK, dtype=jnp.float32), C_IN, axis=1)

    src3 = edge_src.reshape(NW, NCH, CH)
    dst3 = edge_dst.reshape(NW, NCH, CH)
    es8 = edge_scalars.reshape(E8, 128)
    attr8p = edge_attr.reshape(E8, PACK)

    x18 = _gather_x1(node_features, src3).reshape(E8, 128)
    ef = _tc_call(es8, x18, attr8p, w1s, w2s, r, s, a).reshape(N_EDGES, C_OUT)
    zeros = jnp.zeros((N_NODES, C_OUT), jnp.float32)
    partials = _scatter_out(ef, dst3, zeros)
    return _combine(partials)


# speedup vs baseline: 1.4951x; 1.1792x over previous
import functools

import jax
import jax.numpy as jnp
import numpy as np
from jax import lax
from jax.experimental import pallas as pl
from jax.experimental.pallas import tpu as pltpu
from jax.experimental.pallas import tpu_sc as plsc

N_NODES = 10000
N_EDGES = 320000
C_IN = 16
C_OUT = 16
NBASIS = 16
HIDDEN = 256
ACT_NORM = 1.7130613088607788

NC = 2
NS = 16
NW = NC * NS
EPW = N_EDGES // NW
CH = 80
NCH = EPW // CH
GROUP = 5
NG = NCH // GROUP
ROWS_PER_TILE = N_NODES // NS

B_TC = 6400
GRID_TC = N_EDGES // B_TC
PACK = 128 // C_IN
E8 = N_EDGES // PACK
B8 = B_TC // PACK


def _tc_body(es_ref, x1_ref, attr_ref, w1_ref, w2_ref, r_ref, s_ref, a_ref,
             out_ref):
    w1 = w1_ref[...]
    w2 = w2_ref[...]
    r = r_ref[...]
    s = s_ref[...]
    attr_b = jnp.dot(attr_ref[...], a_ref[...],
                     preferred_element_type=jnp.float32)
    for p in range(PACK):
        sl = pl.ds(p * C_IN, C_IN)
        h = jnp.dot(es_ref[:, sl].astype(jnp.bfloat16), w1,
                    preferred_element_type=jnp.float32)
        h = jnp.maximum(h, 0.0) * ACT_NORM
        wt = jnp.dot(h.astype(jnp.bfloat16), w2,
                     preferred_element_type=jnp.float32)
        x1b = jnp.dot(x1_ref[:, sl].astype(jnp.bfloat16), r,
                      preferred_element_type=jnp.float32)
        ef = jnp.dot((wt * x1b).astype(jnp.bfloat16), s,
                     preferred_element_type=jnp.float32)
        out_ref[:, sl] = ef * attr_b[:, p * C_IN:(p + 1) * C_IN]


def _tc_call(es8, x18, attr8p, w1s, w2s, r, s, a):
    return pl.pallas_call(
        _tc_body,
        grid=(GRID_TC,),
        in_specs=[
            pl.BlockSpec((B8, 128), lambda i: (i, 0)),
            pl.BlockSpec((B8, 128), lambda i: (i, 0)),
            pl.BlockSpec((B8, PACK), lambda i: (i, 0)),
            pl.BlockSpec((NBASIS, HIDDEN), lambda i: (0, 0)),
            pl.BlockSpec((HIDDEN, HIDDEN), lambda i: (0, 0)),
            pl.BlockSpec((C_IN, HIDDEN), lambda i: (0, 0)),
            pl.BlockSpec((HIDDEN, C_OUT), lambda i: (0, 0)),
            pl.BlockSpec((PACK, 128), lambda i: (0, 0)),
        ],
        out_specs=pl.BlockSpec((B8, 128), lambda i: (i, 0)),
        out_shape=jax.ShapeDtypeStruct((E8, 128), jnp.float32),
    )(es8, x18, attr8p, w1s, w2s, r, s, a)


def _gather_x1(node_features, src3):
    mesh = plsc.VectorSubcoreMesh(core_axis_name="c", subcore_axis_name="s")

    @functools.partial(
        pl.kernel, mesh=mesh,
        out_type=jax.ShapeDtypeStruct((N_EDGES, C_IN), jnp.float32),
        scratch_types=[
            pltpu.VMEM((NCH, CH), jnp.int32),
            pltpu.VMEM((GROUP, CH, C_IN), jnp.float32),
            pltpu.SemaphoreType.DMA,
            pltpu.SemaphoreType.DMA,
        ],
        compiler_params=pltpu.CompilerParams(use_tc_tiling_on_sc=False),
    )
    def gk(table_hbm, src_hbm, out_hbm, idx_v, rows_v, gsem, wsem):
        wid = lax.axis_index("s") * NC + lax.axis_index("c")
        base = wid * EPW
        pltpu.sync_copy(src_hbm.at[wid], idx_v)

        def body(g, carry):
            j0 = g * GROUP
            cps = [
                pltpu.async_copy(table_hbm.at[idx_v.at[j0 + b]],
                                 rows_v.at[b], gsem)
                for b in range(GROUP)
            ]
            for cp in cps:
                cp.wait()
            wps = [
                pltpu.async_copy(
                    rows_v.at[b],
                    out_hbm.at[pl.ds(base + (j0 + b) * CH, CH)], wsem)
                for b in range(GROUP)
            ]
            for cp in wps:
                cp.wait()
            return carry

        lax.fori_loop(0, NG, body, 0)

    return gk(node_features, src3)


def _scatter_out(ef2, dst3, zeros):
    mesh = plsc.VectorSubcoreMesh(core_axis_name="c", subcore_axis_name="s")

    @functools.partial(
        pl.kernel, mesh=mesh,
        out_type=jax.ShapeDtypeStruct((NC, N_NODES, C_OUT), jnp.float32),
        scratch_types=[
            pltpu.VMEM((NCH, CH), jnp.int32),
            pltpu.VMEM((GROUP, CH, C_OUT), jnp.float32),
            pltpu.VMEM_SHARED((N_NODES, C_OUT), jnp.float32),
            pltpu.SemaphoreType.DMA,
        ],
        compiler_params=pltpu.CompilerParams(use_tc_tiling_on_sc=False),
    )
    def sk(ef_hbm, dst_hbm, z_hbm, out_hbm, idx_v, rows_v, acc, sem):
        c = lax.axis_index("c")
        s = lax.axis_index("s")
        wid = s * NC + c
        base = wid * EPW
        r0 = s * ROWS_PER_TILE
        pltpu.sync_copy(z_hbm.at[pl.ds(r0, ROWS_PER_TILE)],
                        acc.at[pl.ds(r0, ROWS_PER_TILE)])
        pltpu.sync_copy(dst_hbm.at[wid], idx_v)
        plsc.subcore_barrier()

        def body(g, carry):
            j0 = g * GROUP
            cps = [
                pltpu.async_copy(
                    ef_hbm.at[pl.ds(base + (j0 + b) * CH, CH)],
                    rows_v.at[b], sem)
                for b in range(GROUP)
            ]
            for cp in cps:
                cp.wait()
            for b in range(GROUP):
                pltpu.sync_copy(rows_v.at[b], acc.at[idx_v.at[j0 + b]],
                                add=True)
            return carry

        lax.fori_loop(0, NG, body, 0)
        plsc.subcore_barrier()
        pltpu.sync_copy(acc.at[pl.ds(r0, ROWS_PER_TILE)],
                        out_hbm.at[c, pl.ds(r0, ROWS_PER_TILE)])

    return sk(ef2, dst3, zeros)


def _combine_body(p_ref, o_ref):
    o_ref[...] = p_ref[0] + p_ref[1]


def _combine(partials):
    return pl.pallas_call(
        _combine_body,
        out_shape=jax.ShapeDtypeStruct((N_NODES, C_OUT), jnp.float32),
    )(partials)


def kernel(node_features, edge_src, edge_dst, edge_attr, edge_scalars,
           num_neighbors, W1, W2):
    nn = jnp.asarray(num_neighbors, jnp.float32)
    w1s = (W1 * np.float32(1.0 / np.sqrt(NBASIS))).astype(jnp.bfloat16)
    w2s = (W2 * (np.float32(1.0 / (np.sqrt(HIDDEN) * np.sqrt(C_IN)))
                 / jnp.sqrt(nn))).astype(jnp.bfloat16)
    r = jnp.repeat(jnp.eye(C_IN, dtype=jnp.bfloat16), C_OUT, axis=1)
    s = jnp.tile(jnp.eye(C_OUT, dtype=jnp.bfloat16), (C_IN, 1))
    a = jnp.repeat(jnp.eye(PACK, dtype=jnp.float32), C_IN, axis=1)

    src3 = edge_src.reshape(NW, NCH, CH)
    dst3 = edge_dst.reshape(NW, NCH, CH)
    es8 = edge_scalars.reshape(E8, 128)
    attr8p = edge_attr.reshape(E8, PACK)

    x18 = _gather_x1(node_features, src3).reshape(E8, 128)
    ef = _tc_call(es8, x18, attr8p, w1s, w2s, r, s, a).reshape(N_EDGES, C_OUT)
    zeros = jnp.zeros((N_NODES, C_OUT), jnp.float32)
    partials = _scatter_out(ef, dst3, zeros)
    return _combine(partials)
